# Initial kernel scaffold; baseline (speedup 1.0000x reference)
import functools
import jax
import jax.numpy as jnp
from jax import lax
from jax.experimental import pallas as pl
from jax.experimental.pallas import tpu as pltpu
from jax.experimental.pallas import tpu_sc as plsc

N = 10000
D = 128
E = 320000
NC = 2
NS = 16
NW = NC * NS
EPT = E // NW
CH = 80
NCHUNK = EPT // CH
RPT = N // NS


def _sc_body(with_deg, *refs):
    if with_deg:
        (y_hbm, src_hbm, dst_hbm, znd_hbm, zn16_hbm, out_hbm, deg_hbm,
         src_v, dst_v, rows_v, ones_v, acc_sh, deg_sh) = refs
    else:
        (y_hbm, src_hbm, dst_hbm, znd_hbm, out_hbm,
         src_v, dst_v, rows_v, acc_sh) = refs

    c = lax.axis_index("c")
    s = lax.axis_index("s")
    wid = s * NC + c

    pltpu.sync_copy(znd_hbm.at[pl.ds(s * RPT, RPT)],
                    acc_sh.at[pl.ds(s * RPT, RPT)])
    if with_deg:
        pltpu.sync_copy(zn16_hbm.at[pl.ds(s * RPT, RPT)],
                        deg_sh.at[pl.ds(s * RPT, RPT)])
        one_row = jnp.where(lax.iota(jnp.int32, 16) == 0, 1.0, 0.0)

        def fill(i, _):
            ones_v[i, :] = one_row
            return ()
        lax.fori_loop(0, CH, fill, ())
    plsc.subcore_barrier()

    base = wid * EPT

    def chunk(i, _):
        off = base + i * CH
        pltpu.sync_copy(src_hbm.at[pl.ds(off, CH)], src_v)
        pltpu.sync_copy(dst_hbm.at[pl.ds(off, CH)], dst_v)
        pltpu.sync_copy(y_hbm.at[src_v], rows_v)
        pltpu.sync_copy(rows_v, acc_sh.at[dst_v], add=True)
        if with_deg:
            pltpu.sync_copy(ones_v, deg_sh.at[dst_v], add=True)
        return ()

    lax.fori_loop(0, NCHUNK, chunk, ())
    plsc.subcore_barrier()

    pltpu.sync_copy(acc_sh.at[pl.ds(s * RPT, RPT)],
                    out_hbm.at[pl.ds(c * N + s * RPT, RPT)])
    if with_deg:
        pltpu.sync_copy(deg_sh.at[pl.ds(s * RPT, RPT)],
                        deg_hbm.at[pl.ds(c * N + s * RPT, RPT)])


def _make_sc_agg(with_deg):
    mesh = plsc.VectorSubcoreMesh(core_axis_name="c", subcore_axis_name="s")
    if with_deg:
        out_type = (jax.ShapeDtypeStruct((NC * N, D), jnp.float32),
                    jax.ShapeDtypeStruct((NC * N, 16), jnp.float32))
    else:
        out_type = jax.ShapeDtypeStruct((NC * N, D), jnp.float32)
    scratch = [
        pltpu.VMEM((CH,), jnp.int32),
        pltpu.VMEM((CH,), jnp.int32),
        pltpu.VMEM((CH, D), jnp.float32),
    ]
    if with_deg:
        scratch.append(pltpu.VMEM((CH, 16), jnp.float32))
    scratch.append(pltpu.VMEM_SHARED((N, D), jnp.float32))
    if with_deg:
        scratch.append(pltpu.VMEM_SHARED((N, 16), jnp.float32))
    return pl.kernel(
        functools.partial(_sc_body, with_deg),
        out_type=out_type,
        mesh=mesh,
        scratch_types=scratch,
    )


BN = 1000


def _tc_pre_body(x_ref, wn_ref, ws_ref, b_ref, y_ref, s_ref):
    x = x_ref[...]
    y_ref[...] = jnp.dot(x, wn_ref[...], preferred_element_type=jnp.float32)
    s_ref[...] = (jnp.dot(x, ws_ref[...], preferred_element_type=jnp.float32)
                  + b_ref[...])


def _tc_mid_body(s1_ref, a0_ref, a1_ref, d0_ref, d1_ref, wn_ref, ws_ref,
                 b_ref, y_ref, s_ref):
    deg = d0_ref[...] + d1_ref[...]
    inv = 1.0 / jnp.maximum(deg[:, 0:1], 1.0)
    h = jnp.maximum(s1_ref[...] + (a0_ref[...] + a1_ref[...]) * inv, 0.0)
    y_ref[...] = jnp.dot(h, wn_ref[...], preferred_element_type=jnp.float32)
    s_ref[...] = (jnp.dot(h, ws_ref[...], preferred_element_type=jnp.float32)
                  + b_ref[...])


def _tc_post_body(s2_ref, a0_ref, a1_ref, d0_ref, d1_ref, out_ref):
    deg = d0_ref[...] + d1_ref[...]
    inv = 1.0 / jnp.maximum(deg[:, 0:1], 1.0)
    out_ref[...] = s2_ref[...] + (a0_ref[...] + a1_ref[...]) * inv


def _row_spec(cols):
    return pl.BlockSpec((BN, cols), lambda i: (i, 0))


def _full_spec(shape):
    return pl.BlockSpec(shape, lambda i: tuple(0 for _ in shape))


_tc_pre = pl.pallas_call(
    _tc_pre_body,
    grid=(N // BN,),
    in_specs=[_row_spec(D), _full_spec((D, D)), _full_spec((D, D)),
              _full_spec((1, D))],
    out_specs=[_row_spec(D), _row_spec(D)],
    out_shape=[jax.ShapeDtypeStruct((N, D), jnp.float32),
               jax.ShapeDtypeStruct((N, D), jnp.float32)],
)

_tc_mid = pl.pallas_call(
    _tc_mid_body,
    grid=(N // BN,),
    in_specs=[_row_spec(D), _row_spec(D), _row_spec(D), _row_spec(16),
              _row_spec(16), _full_spec((D, D)), _full_spec((D, D)),
              _full_spec((1, D))],
    out_specs=[_row_spec(D), _row_spec(D)],
    out_shape=[jax.ShapeDtypeStruct((N, D), jnp.float32),
               jax.ShapeDtypeStruct((N, D), jnp.float32)],
)

_tc_post = pl.pallas_call(
    _tc_post_body,
    grid=(N // BN,),
    in_specs=[_row_spec(D), _row_spec(D), _row_spec(D), _row_spec(16),
              _row_spec(16)],
    out_specs=_row_spec(D),
    out_shape=jax.ShapeDtypeStruct((N, D), jnp.float32),
)

_sc_agg_deg = _make_sc_agg(True)
_sc_agg = _make_sc_agg(False)


@jax.jit
def kernel(in_feat, edge_index, W_self1, W_neigh1, b1, W_self2, W_neigh2, b2):
    src = edge_index[0].astype(jnp.int32)
    dst = edge_index[1].astype(jnp.int32)
    znd = jnp.zeros((N, D), jnp.float32)
    zn16 = jnp.zeros((N, 16), jnp.float32)

    y1, s1 = _tc_pre(in_feat, W_neigh1, W_self1, b1.reshape(1, D))
    agg1, deg = _sc_agg_deg(y1, src, dst, znd, zn16)
    y2, s2 = _tc_mid(s1, agg1[:N], agg1[N:], deg[:N], deg[N:],
                     W_neigh2, W_self2, b2.reshape(1, D))
    agg2 = _sc_agg(y2, src, dst, znd)
    return _tc_post(s2, agg2[:N], agg2[N:], deg[:N], deg[N:])


# speedup vs baseline: 3.1401x; 3.1401x over previous
import functools
import jax
import jax.numpy as jnp
from jax import lax
from jax.experimental import pallas as pl
from jax.experimental.pallas import tpu as pltpu
from jax.experimental.pallas import tpu_sc as plsc

N = 10000
D = 128
E = 320000
NC = 2
NS = 16
NW = NC * NS
CH = 128
EP = 327680
EPT = EP // NW
NCHUNK = EPT // CH
NP = 10240
RPT = NP // NS
L = 16


def _sc_body(with_deg, *refs):
    if with_deg:
        (y_hbm, src_hbm, dst_hbm, z_hbm, out_hbm, deg_hbm,
         src_v, dst_v, rows_v, degt_v, acc_sh, sem) = refs
    else:
        (y_hbm, src_hbm, dst_hbm, z_hbm, out_hbm,
         src_v, dst_v, rows_v, acc_sh, sem) = refs

    c = lax.axis_index("c")
    s = lax.axis_index("s")
    wid = s * NC + c

    def zc(q, _):
        r = s * RPT + q * CH
        pltpu.sync_copy(z_hbm.at[pl.ds(r, CH)], rows_v)
        pltpu.sync_copy(rows_v, acc_sh.at[pl.ds(r, CH)])
        return ()
    lax.fori_loop(0, RPT // CH, zc, ())

    if with_deg:
        zv = jnp.zeros((L,), jnp.float32)

        def zd(q, _):
            degt_v[pl.ds(q * L, L)] = zv
            return ()
        lax.fori_loop(0, NP // L, zd, ())
    plsc.subcore_barrier()

    base = wid * EPT
    if with_deg:
        onev = jnp.ones((L,), jnp.float32)

    def chunk(i, _):
        off = base + i * CH
        pltpu.sync_copy(src_hbm.at[pl.ds(off, CH)], src_v)
        pltpu.sync_copy(dst_hbm.at[pl.ds(off, CH)], dst_v)
        pltpu.async_copy(y_hbm.at[src_v], rows_v, sem).wait()
        pltpu.sync_copy(rows_v, acc_sh.at[dst_v], add=True)
        if with_deg:
            for k in range(CH // L):
                idx = dst_v[pl.ds(k * L, L)]
                plsc.addupdate_scatter(degt_v, [idx], onev)
        return ()

    lax.fori_loop(0, NCHUNK, chunk, ())
    plsc.subcore_barrier()

    def oc(q, _):
        r = s * RPT + q * CH
        pltpu.sync_copy(acc_sh.at[pl.ds(r, CH)], rows_v)
        pltpu.sync_copy(rows_v, out_hbm.at[pl.ds(c * NP + r, CH)])
        return ()
    lax.fori_loop(0, RPT // CH, oc, ())
    if with_deg:
        pltpu.sync_copy(degt_v, deg_hbm.at[wid])


def _make_sc_agg(with_deg):
    mesh = plsc.VectorSubcoreMesh(core_axis_name="c", subcore_axis_name="s")
    if with_deg:
        out_type = (jax.ShapeDtypeStruct((NC * NP, D), jnp.float32),
                    jax.ShapeDtypeStruct((NW, NP), jnp.float32))
    else:
        out_type = jax.ShapeDtypeStruct((NC * NP, D), jnp.float32)
    scratch = [
        pltpu.VMEM((CH,), jnp.int32),
        pltpu.VMEM((CH,), jnp.int32),
        pltpu.VMEM((CH, D), jnp.float32),
    ]
    if with_deg:
        scratch.append(pltpu.VMEM((NP,), jnp.float32))
    scratch.append(pltpu.VMEM_SHARED((NP, D), jnp.float32))
    scratch.append(pltpu.SemaphoreType.DMA)
    return pl.kernel(
        functools.partial(_sc_body, with_deg),
        out_type=out_type,
        mesh=mesh,
        scratch_types=scratch,
        compiler_params=pltpu.CompilerParams(needs_layout_passes=False),
    )


BN = 1000


def _tc_pre_body(x_ref, wn_ref, ws_ref, b_ref, y_ref, s_ref):
    x = x_ref[...]
    y_ref[...] = jnp.dot(x, wn_ref[...], preferred_element_type=jnp.float32)
    s_ref[...] = (jnp.dot(x, ws_ref[...], preferred_element_type=jnp.float32)
                  + b_ref[...])


def _tc_mid_body(s1_ref, a0_ref, a1_ref, dp_ref, wn_ref, ws_ref, b_ref,
                 y_ref, s_ref, i_ref):
    deg = jnp.sum(dp_ref[...], axis=1)[:, None]
    inv = 1.0 / jnp.maximum(deg, 1.0)
    h = jnp.maximum(s1_ref[...] + (a0_ref[...] + a1_ref[...]) * inv, 0.0)
    y_ref[...] = jnp.dot(h, wn_ref[...], preferred_element_type=jnp.float32)
    s_ref[...] = (jnp.dot(h, ws_ref[...], preferred_element_type=jnp.float32)
                  + b_ref[...])
    i_ref[...] = jnp.broadcast_to(inv, (BN, D))


def _tc_post_body(s2_ref, a0_ref, a1_ref, i_ref, out_ref):
    out_ref[...] = s2_ref[...] + (a0_ref[...] + a1_ref[...]) * i_ref[...]


def _row_spec(cols):
    return pl.BlockSpec((BN, cols), lambda i: (i, 0))


def _full_spec(shape):
    return pl.BlockSpec(shape, lambda i: tuple(0 for _ in shape))


_tc_pre = pl.pallas_call(
    _tc_pre_body,
    grid=(N // BN,),
    in_specs=[_row_spec(D), _full_spec((D, D)), _full_spec((D, D)),
              _full_spec((1, D))],
    out_specs=[_row_spec(D), _row_spec(D)],
    out_shape=[jax.ShapeDtypeStruct((N, D), jnp.float32),
               jax.ShapeDtypeStruct((N, D), jnp.float32)],
)

_tc_mid = pl.pallas_call(
    _tc_mid_body,
    grid=(N // BN,),
    in_specs=[_row_spec(D), _row_spec(D), _row_spec(D),
              pl.BlockSpec((BN, NW), lambda i: (i, 0)),
              _full_spec((D, D)), _full_spec((D, D)), _full_spec((1, D))],
    out_specs=[_row_spec(D), _row_spec(D), _row_spec(D)],
    out_shape=[jax.ShapeDtypeStruct((N, D), jnp.float32),
               jax.ShapeDtypeStruct((N, D), jnp.float32),
               jax.ShapeDtypeStruct((N, D), jnp.float32)],
)

_tc_post = pl.pallas_call(
    _tc_post_body,
    grid=(N // BN,),
    in_specs=[_row_spec(D), _row_spec(D), _row_spec(D), _row_spec(D)],
    out_specs=_row_spec(D),
    out_shape=jax.ShapeDtypeStruct((N, D), jnp.float32),
)

_sc_agg_deg = _make_sc_agg(True)
_sc_agg = _make_sc_agg(False)


@jax.jit
def kernel(in_feat, edge_index, W_self1, W_neigh1, b1, W_self2, W_neigh2, b2):
    pad = EP - E
    src = jnp.concatenate(
        [edge_index[0].astype(jnp.int32), jnp.zeros((pad,), jnp.int32)])
    dst = jnp.concatenate(
        [edge_index[1].astype(jnp.int32), jnp.full((pad,), N, jnp.int32)])
    znd = jnp.zeros((NP, D), jnp.float32)

    y1, s1 = _tc_pre(in_feat, W_neigh1, W_self1, b1.reshape(1, D))
    agg1, degp = _sc_agg_deg(y1, src, dst, znd)
    y2, s2, inv = _tc_mid(s1, agg1[:N], agg1[NP:NP + N], degp.T,
                          W_neigh2, W_self2, b2.reshape(1, D))
    agg2 = _sc_agg(y2, src, dst, znd)
    return _tc_post(s2, agg2[:N], agg2[NP:NP + N], inv)


# speedup vs baseline: 3.9680x; 1.2637x over previous
import functools
import jax
import jax.numpy as jnp
from jax import lax
from jax.experimental import pallas as pl
from jax.experimental.pallas import tpu as pltpu
from jax.experimental.pallas import tpu_sc as plsc

N = 10000
D = 128
E = 320000
NC = 2
NS = 16
NW = NC * NS
CH = 128
EP = 327680
EPT = EP // NW
NCHUNK = EPT // CH
NP = 10240
RPT = NP // NS
L = 16


def _sc_body(with_deg, *refs):
    if with_deg:
        (y_hbm, src_hbm, dst_hbm, z_hbm, out_hbm, deg_hbm,
         src_a, dst_a, src_b, dst_b, rows_a, rows_b, degt_v,
         acc_sh, sema, semb) = refs
    else:
        (y_hbm, src_hbm, dst_hbm, z_hbm, out_hbm,
         src_a, dst_a, src_b, dst_b, rows_a, rows_b,
         acc_sh, sema, semb) = refs
    rows_v = rows_a

    c = lax.axis_index("c")
    s = lax.axis_index("s")
    wid = s * NC + c

    def zc(q, _):
        r = s * RPT + q * CH
        pltpu.sync_copy(z_hbm.at[pl.ds(r, CH)], rows_v)
        pltpu.sync_copy(rows_v, acc_sh.at[pl.ds(r, CH)])
        return ()
    lax.fori_loop(0, RPT // CH, zc, ())

    if with_deg:
        zv = jnp.zeros((L,), jnp.float32)

        def zd(q, _):
            degt_v[pl.ds(q * L, L)] = zv
            return ()
        lax.fori_loop(0, NP // L, zd, ())
    plsc.subcore_barrier()

    base = wid * EPT
    if with_deg:
        onev = jnp.ones((L,), jnp.float32)

    def load_idx(o, sv, dv):
        pltpu.sync_copy(src_hbm.at[pl.ds(o, CH)], sv)
        pltpu.sync_copy(dst_hbm.at[pl.ds(o, CH)], dv)

    def bump_deg(dv):
        if with_deg:
            for k in range(CH // L):
                plsc.addupdate_scatter(degt_v, [dv[pl.ds(k * L, L)]], onev)

    load_idx(base, src_a, dst_a)
    pltpu.async_copy(y_hbm.at[src_a], rows_a, sema)

    def pair(j, _):
        o1 = base + (2 * j + 1) * CH
        load_idx(o1, src_b, dst_b)
        pltpu.async_copy(y_hbm.at[src_b], rows_b, semb)
        pltpu.make_async_copy(y_hbm.at[src_a], rows_a, sema).wait()
        pltpu.sync_copy(rows_a, acc_sh.at[dst_a], add=True)
        bump_deg(dst_a)
        o2 = base + (2 * j + 2) * CH
        load_idx(o2, src_a, dst_a)
        pltpu.async_copy(y_hbm.at[src_a], rows_a, sema)
        pltpu.make_async_copy(y_hbm.at[src_b], rows_b, semb).wait()
        pltpu.sync_copy(rows_b, acc_sh.at[dst_b], add=True)
        bump_deg(dst_b)
        return ()

    lax.fori_loop(0, NCHUNK // 2 - 1, pair, ())
    o1 = base + (NCHUNK - 1) * CH
    load_idx(o1, src_b, dst_b)
    pltpu.async_copy(y_hbm.at[src_b], rows_b, semb)
    pltpu.make_async_copy(y_hbm.at[src_a], rows_a, sema).wait()
    pltpu.sync_copy(rows_a, acc_sh.at[dst_a], add=True)
    bump_deg(dst_a)
    pltpu.make_async_copy(y_hbm.at[src_b], rows_b, semb).wait()
    pltpu.sync_copy(rows_b, acc_sh.at[dst_b], add=True)
    bump_deg(dst_b)
    plsc.subcore_barrier()

    def oc(q, _):
        r = s * RPT + q * CH
        pltpu.sync_copy(acc_sh.at[pl.ds(r, CH)], rows_v)
        pltpu.sync_copy(rows_v, out_hbm.at[pl.ds(c * NP + r, CH)])
        return ()
    lax.fori_loop(0, RPT // CH, oc, ())
    if with_deg:
        pltpu.sync_copy(degt_v, deg_hbm.at[wid])


def _make_sc_agg(with_deg):
    mesh = plsc.VectorSubcoreMesh(core_axis_name="c", subcore_axis_name="s")
    if with_deg:
        out_type = (jax.ShapeDtypeStruct((NC * NP, D), jnp.float32),
                    jax.ShapeDtypeStruct((NW, NP), jnp.float32))
    else:
        out_type = jax.ShapeDtypeStruct((NC * NP, D), jnp.float32)
    scratch = [
        pltpu.VMEM((CH,), jnp.int32),
        pltpu.VMEM((CH,), jnp.int32),
        pltpu.VMEM((CH,), jnp.int32),
        pltpu.VMEM((CH,), jnp.int32),
        pltpu.VMEM((CH, D), jnp.float32),
        pltpu.VMEM((CH, D), jnp.float32),
    ]
    if with_deg:
        scratch.append(pltpu.VMEM((NP,), jnp.float32))
    scratch.append(pltpu.VMEM_SHARED((NP, D), jnp.float32))
    scratch.append(pltpu.SemaphoreType.DMA)
    scratch.append(pltpu.SemaphoreType.DMA)
    return pl.kernel(
        functools.partial(_sc_body, with_deg),
        out_type=out_type,
        mesh=mesh,
        scratch_types=scratch,
        compiler_params=pltpu.CompilerParams(needs_layout_passes=False),
    )


BN = 1000


def _tc_pre_body(x_ref, wn_ref, ws_ref, b_ref, y_ref, s_ref):
    x = x_ref[...]
    y_ref[...] = jnp.dot(x, wn_ref[...], preferred_element_type=jnp.float32)
    s_ref[...] = (jnp.dot(x, ws_ref[...], preferred_element_type=jnp.float32)
                  + b_ref[...])


def _tc_mid_body(s1_ref, a0_ref, a1_ref, dp_ref, wn_ref, ws_ref, b_ref,
                 y_ref, s_ref, i_ref):
    deg = jnp.sum(dp_ref[...], axis=1)[:, None]
    inv = 1.0 / jnp.maximum(deg, 1.0)
    h = jnp.maximum(s1_ref[...] + (a0_ref[...] + a1_ref[...]) * inv, 0.0)
    y_ref[...] = jnp.dot(h, wn_ref[...], preferred_element_type=jnp.float32)
    s_ref[...] = (jnp.dot(h, ws_ref[...], preferred_element_type=jnp.float32)
                  + b_ref[...])
    i_ref[...] = jnp.broadcast_to(inv, (BN, D))


def _tc_post_body(s2_ref, a0_ref, a1_ref, i_ref, out_ref):
    out_ref[...] = s2_ref[...] + (a0_ref[...] + a1_ref[...]) * i_ref[...]


def _row_spec(cols):
    return pl.BlockSpec((BN, cols), lambda i: (i, 0))


def _full_spec(shape):
    return pl.BlockSpec(shape, lambda i: tuple(0 for _ in shape))


_tc_pre = pl.pallas_call(
    _tc_pre_body,
    grid=(N // BN,),
    in_specs=[_row_spec(D), _full_spec((D, D)), _full_spec((D, D)),
              _full_spec((1, D))],
    out_specs=[_row_spec(D), _row_spec(D)],
    out_shape=[jax.ShapeDtypeStruct((N, D), jnp.float32),
               jax.ShapeDtypeStruct((N, D), jnp.float32)],
)

_tc_mid = pl.pallas_call(
    _tc_mid_body,
    grid=(N // BN,),
    in_specs=[_row_spec(D), _row_spec(D), _row_spec(D),
              pl.BlockSpec((BN, NW), lambda i: (i, 0)),
              _full_spec((D, D)), _full_spec((D, D)), _full_spec((1, D))],
    out_specs=[_row_spec(D), _row_spec(D), _row_spec(D)],
    out_shape=[jax.ShapeDtypeStruct((N, D), jnp.float32),
               jax.ShapeDtypeStruct((N, D), jnp.float32),
               jax.ShapeDtypeStruct((N, D), jnp.float32)],
)

_tc_post = pl.pallas_call(
    _tc_post_body,
    grid=(N // BN,),
    in_specs=[_row_spec(D), _row_spec(D), _row_spec(D), _row_spec(D)],
    out_specs=_row_spec(D),
    out_shape=jax.ShapeDtypeStruct((N, D), jnp.float32),
)

_sc_agg_deg = _make_sc_agg(True)
_sc_agg = _make_sc_agg(False)


@jax.jit
def kernel(in_feat, edge_index, W_self1, W_neigh1, b1, W_self2, W_neigh2, b2):
    pad = EP - E
    src = jnp.concatenate(
        [edge_index[0].astype(jnp.int32), jnp.zeros((pad,), jnp.int32)])
    dst = jnp.concatenate(
        [edge_index[1].astype(jnp.int32), jnp.full((pad,), N, jnp.int32)])
    znd = jnp.zeros((NP, D), jnp.float32)

    y1, s1 = _tc_pre(in_feat, W_neigh1, W_self1, b1.reshape(1, D))
    agg1, degp = _sc_agg_deg(y1, src, dst, znd)
    y2, s2, inv = _tc_mid(s1, agg1[:N], agg1[NP:NP + N], degp.T,
                          W_neigh2, W_self2, b2.reshape(1, D))
    agg2 = _sc_agg(y2, src, dst, znd)
    return _tc_post(s2, agg2[:N], agg2[NP:NP + N], inv)
